# initial kernel scaffold (unmeasured)
import jax
import jax.numpy as jnp
from jax import lax
from jax.experimental import pallas as pl
from jax.experimental.pallas import tpu as pltpu

N_DEV = 16
N_STEPS = 4
N_LAYERS = 3
N_SLOTS = N_LAYERS * N_STEPS


def kernel(x, Win0, Wout0, Win1, Wout1, Win2, Wout2):
    b, d_shard = x.shape
    h_dim = Win0.shape[1]

    def body(x_ref, win0_ref, wout0_ref, win1_ref, wout1_ref,
             win2_ref, wout2_ref, out_ref,
             send_ref, recv_ref, send_sems, recv_sems):
        my_pos = lax.axis_index("i")

        xv = x_ref[...].astype(jnp.bfloat16)

        wins = [win0_ref, win1_ref, win2_ref]
        wouts = [wout0_ref, wout1_ref, wout2_ref]

        for layer in range(N_LAYERS):
            acc = jnp.dot(
                xv, wins[layer][...].astype(jnp.bfloat16),
                preferred_element_type=jnp.float32,
            )

            for s in range(N_STEPS):
                slot = layer * N_STEPS + s
                partner = jnp.bitwise_xor(my_pos, 1 << s)
                send_ref[...] = acc.astype(jnp.bfloat16)
                rdma = pltpu.make_async_remote_copy(
                    src_ref=send_ref,
                    dst_ref=recv_ref.at[slot],
                    send_sem=send_sems.at[slot],
                    recv_sem=recv_sems.at[slot],
                    device_id=(partner,),
                    device_id_type=pl.DeviceIdType.MESH,
                )
                rdma.start()
                rdma.wait()
                acc = acc + recv_ref[slot].astype(jnp.float32)

            h = jnp.maximum(acc, 0.0).astype(jnp.bfloat16)
            nxt = jnp.dot(
                h, wouts[layer][...].astype(jnp.bfloat16),
                preferred_element_type=jnp.float32,
            )
            if layer < N_LAYERS - 1:
                xv = nxt.astype(jnp.bfloat16)
            else:
                out_ref[...] = nxt

    return pl.pallas_call(
        body,
        out_shape=jax.ShapeDtypeStruct((b, d_shard), jnp.float32),
        in_specs=[pl.BlockSpec(memory_space=pltpu.VMEM)] * 7,
        out_specs=pl.BlockSpec(memory_space=pltpu.VMEM),
        scratch_shapes=[
            pltpu.VMEM((b, h_dim), jnp.bfloat16),
            pltpu.VMEM((N_SLOTS, b, h_dim), jnp.bfloat16),
            pltpu.SemaphoreType.DMA((N_SLOTS,)),
            pltpu.SemaphoreType.DMA((N_SLOTS,)),
        ],
        compiler_params=pltpu.CompilerParams(collective_id=0),
    )(x, Win0, Wout0, Win1, Wout1, Win2, Wout2)


# baseline (device time: 83972 ns/iter reference)
import jax
import jax.numpy as jnp
from jax import lax
from jax.experimental import pallas as pl
from jax.experimental.pallas import tpu as pltpu

N_DEV = 16
N_STEPS = 4
N_LAYERS = 3
N_SLOTS = N_LAYERS * N_STEPS


def kernel(x, Win0, Wout0, Win1, Wout1, Win2, Wout2):
    b, d_shard = x.shape
    h_dim = Win0.shape[1]

    def body(x_ref, win0_ref, wout0_ref, win1_ref, wout1_ref,
             win2_ref, wout2_ref, out_ref,
             send_ref, recv_ref, send_sems, recv_sems):
        my_pos = lax.axis_index("i")

        xv = x_ref[...].astype(jnp.bfloat16)

        wins = [win0_ref, win1_ref, win2_ref]
        wouts = [wout0_ref, wout1_ref, wout2_ref]

        for layer in range(N_LAYERS):
            acc = jnp.dot(
                xv, wins[layer][...].astype(jnp.bfloat16),
                preferred_element_type=jnp.float32,
            )

            for s in range(N_STEPS):
                slot = layer * N_STEPS + s
                partner = jnp.bitwise_xor(my_pos, 1 << s)
                send_ref[...] = acc.astype(jnp.bfloat16)
                rdma = pltpu.make_async_remote_copy(
                    src_ref=send_ref,
                    dst_ref=recv_ref.at[slot],
                    send_sem=send_sems.at[slot],
                    recv_sem=recv_sems.at[slot],
                    device_id=(partner,),
                    device_id_type=pl.DeviceIdType.MESH,
                )
                rdma.start()
                rdma.wait()
                acc = acc + recv_ref[slot].astype(jnp.float32)

            h = jnp.maximum(acc, 0.0).astype(jnp.bfloat16)
            nxt = jnp.dot(
                h, wouts[layer][...].astype(jnp.bfloat16),
                preferred_element_type=jnp.float32,
            )
            if layer < N_LAYERS - 1:
                xv = nxt.astype(jnp.bfloat16)
            else:
                out_ref[...] = nxt

    return pl.pallas_call(
        body,
        out_shape=jax.ShapeDtypeStruct((b, d_shard), jnp.float32),
        in_specs=[pl.BlockSpec(memory_space=pltpu.VMEM)] * 7,
        out_specs=pl.BlockSpec(memory_space=pltpu.VMEM),
        scratch_shapes=[
            pltpu.VMEM((b, h_dim), jnp.bfloat16),
            pltpu.VMEM((N_SLOTS, b, h_dim), jnp.bfloat16),
            pltpu.SemaphoreType.DMA((N_SLOTS,)),
            pltpu.SemaphoreType.DMA((N_SLOTS,)),
        ],
    )(x, Win0, Wout0, Win1, Wout1, Win2, Wout2)


# device time: 49778 ns/iter; 1.6869x vs baseline; 1.6869x over previous
import jax
import jax.numpy as jnp
from jax import lax
from jax.experimental import pallas as pl
from jax.experimental.pallas import tpu as pltpu

N_DEV = 16
N_LAYERS = 3
CHUNK = 16


def kernel(x, Win0, Wout0, Win1, Wout1, Win2, Wout2):
    b, d_shard = x.shape
    h_dim = Win0.shape[1]

    def body(x_ref, win0_ref, wout0_ref, win1_ref, wout1_ref,
             win2_ref, wout2_ref, out_ref,
             part_ref, rs_ref, h_ref,
             rs_send_sems, rs_recv_sems, ag_send_sems, ag_recv_sems):
        me = lax.axis_index("i")

        xv = x_ref[...].astype(jnp.bfloat16)

        wins = [win0_ref, win1_ref, win2_ref]
        wouts = [wout0_ref, wout1_ref, wout2_ref]

        for layer in range(N_LAYERS):
            partial = jnp.dot(
                xv, wins[layer][...].astype(jnp.bfloat16),
                preferred_element_type=jnp.float32,
            )
            part_ref[...] = partial.astype(jnp.bfloat16)
            rs_ref[me] = part_ref[pl.ds(me * CHUNK, CHUNK)]

            rs_sends = []
            for d in range(1, N_DEV):
                q = lax.rem(me + d, N_DEV)
                rdma = pltpu.make_async_remote_copy(
                    src_ref=part_ref.at[pl.ds(q * CHUNK, CHUNK)],
                    dst_ref=rs_ref.at[me],
                    send_sem=rs_send_sems.at[q],
                    recv_sem=rs_recv_sems.at[me],
                    device_id=(q,),
                    device_id_type=pl.DeviceIdType.MESH,
                )
                rdma.start()
                rs_sends.append(rdma)
            for d in range(1, N_DEV):
                p = lax.rem(me + d, N_DEV)
                pltpu.make_async_remote_copy(
                    src_ref=part_ref.at[pl.ds(0, CHUNK)],
                    dst_ref=rs_ref.at[p],
                    send_sem=rs_send_sems.at[p],
                    recv_sem=rs_recv_sems.at[p],
                    device_id=(p,),
                    device_id_type=pl.DeviceIdType.MESH,
                ).wait_recv()
            for rdma in rs_sends:
                rdma.wait_send()

            total = jnp.sum(rs_ref[...].astype(jnp.float32), axis=0)
            h_chunk = jnp.maximum(total, 0.0).astype(jnp.bfloat16)
            h_ref[pl.ds(me * CHUNK, CHUNK)] = h_chunk

            ag_sends = []
            for d in range(1, N_DEV):
                q = lax.rem(me + d, N_DEV)
                rdma = pltpu.make_async_remote_copy(
                    src_ref=h_ref.at[pl.ds(me * CHUNK, CHUNK)],
                    dst_ref=h_ref.at[pl.ds(me * CHUNK, CHUNK)],
                    send_sem=ag_send_sems.at[q],
                    recv_sem=ag_recv_sems.at[me],
                    device_id=(q,),
                    device_id_type=pl.DeviceIdType.MESH,
                )
                rdma.start()
                ag_sends.append(rdma)
            for d in range(1, N_DEV):
                p = lax.rem(me + d, N_DEV)
                pltpu.make_async_remote_copy(
                    src_ref=h_ref.at[pl.ds(p * CHUNK, CHUNK)],
                    dst_ref=h_ref.at[pl.ds(p * CHUNK, CHUNK)],
                    send_sem=ag_send_sems.at[p],
                    recv_sem=ag_recv_sems.at[p],
                    device_id=(p,),
                    device_id_type=pl.DeviceIdType.MESH,
                ).wait_recv()
            for rdma in ag_sends:
                rdma.wait_send()

            nxt = jnp.dot(
                h_ref[...], wouts[layer][...].astype(jnp.bfloat16),
                preferred_element_type=jnp.float32,
            )
            if layer < N_LAYERS - 1:
                xv = nxt.astype(jnp.bfloat16)
            else:
                out_ref[...] = nxt

    return pl.pallas_call(
        body,
        out_shape=jax.ShapeDtypeStruct((b, d_shard), jnp.float32),
        in_specs=[pl.BlockSpec(memory_space=pltpu.VMEM)] * 7,
        out_specs=pl.BlockSpec(memory_space=pltpu.VMEM),
        scratch_shapes=[
            pltpu.VMEM((b, h_dim), jnp.bfloat16),
            pltpu.VMEM((N_DEV, CHUNK, h_dim), jnp.bfloat16),
            pltpu.VMEM((b, h_dim), jnp.bfloat16),
            pltpu.SemaphoreType.DMA((N_DEV,)),
            pltpu.SemaphoreType.DMA((N_DEV,)),
            pltpu.SemaphoreType.DMA((N_DEV,)),
            pltpu.SemaphoreType.DMA((N_DEV,)),
        ],
    )(x, Win0, Wout0, Win1, Wout1, Win2, Wout2)


# device time: 49696 ns/iter; 1.6897x vs baseline; 1.0017x over previous
import jax
import jax.numpy as jnp
from jax import lax
from jax.experimental import pallas as pl
from jax.experimental.pallas import tpu as pltpu

N_DEV = 16
N_LAYERS = 3
CHUNK = 16
GROUPS = [range(0, 8), range(8, 16)]


def kernel(x, Win0, Wout0, Win1, Wout1, Win2, Wout2):
    b, d_shard = x.shape
    h_dim = Win0.shape[1]

    def body(x_ref, win0_ref, wout0_ref, win1_ref, wout1_ref,
             win2_ref, wout2_ref, out_ref,
             part_ref, rs_ref, h_ref, ag_src_ref,
             rs_send_sems, rs_recv_sems, ag_send_sems, ag_recv_sems):
        me = lax.axis_index("i")

        wins = [win0_ref, win1_ref, win2_ref]
        wouts = [wout0_ref, wout1_ref, wout2_ref]

        def rs_send(q):
            rdma = pltpu.make_async_remote_copy(
                src_ref=part_ref.at[pl.ds(q * CHUNK, CHUNK)],
                dst_ref=rs_ref.at[me],
                send_sem=rs_send_sems.at[q],
                recv_sem=rs_recv_sems.at[me],
                device_id=(q,),
                device_id_type=pl.DeviceIdType.MESH,
            )
            rdma.start()
            return rdma

        def rs_recv_wait(p):
            pltpu.make_async_remote_copy(
                src_ref=part_ref.at[pl.ds(0, CHUNK)],
                dst_ref=rs_ref.at[p],
                send_sem=rs_send_sems.at[p],
                recv_sem=rs_recv_sems.at[p],
                device_id=(p,),
                device_id_type=pl.DeviceIdType.MESH,
            ).wait_recv()

        def ag_send(q):
            rdma = pltpu.make_async_remote_copy(
                src_ref=ag_src_ref,
                dst_ref=h_ref.at[pl.ds(me * CHUNK, CHUNK)],
                send_sem=ag_send_sems.at[q],
                recv_sem=ag_recv_sems.at[me],
                device_id=(q,),
                device_id_type=pl.DeviceIdType.MESH,
            )
            rdma.start()
            return rdma

        def ag_recv_wait(p):
            pltpu.make_async_remote_copy(
                src_ref=ag_src_ref,
                dst_ref=h_ref.at[pl.ds(p * CHUNK, CHUNK)],
                send_sem=ag_send_sems.at[p],
                recv_sem=ag_recv_sems.at[p],
                device_id=(p,),
                device_id_type=pl.DeviceIdType.MESH,
            ).wait_recv()

        xv = x_ref[...].astype(jnp.bfloat16)
        part_ref[...] = jnp.dot(
            xv, wins[0][...].astype(jnp.bfloat16),
            preferred_element_type=jnp.float32,
        ).astype(jnp.bfloat16)
        pending_rs = [rs_send(q) for q in range(N_DEV)]
        pending_ag = []

        for layer in range(N_LAYERS):
            for p in range(N_DEV):
                rs_recv_wait(p)
            for rdma in pending_rs:
                rdma.wait_send()
            for rdma in pending_ag:
                rdma.wait_send()

            total = jnp.sum(rs_ref[...].astype(jnp.float32), axis=0)
            ag_src_ref[...] = jnp.maximum(total, 0.0).astype(jnp.bfloat16)

            pending_ag = [ag_send(q) for q in range(N_DEV)]

            if layer < N_LAYERS - 1:
                w_out = wouts[layer][...].astype(jnp.bfloat16)
                w_in = wins[layer + 1][...].astype(jnp.bfloat16)
                pending_rs = []
                for grp in GROUPS:
                    for p in grp:
                        ag_recv_wait(p)
                    rows = pl.ds(grp.start * CHUNK, len(grp) * CHUNK)
                    xblk = jnp.dot(
                        h_ref[rows, :], w_out,
                        preferred_element_type=jnp.float32,
                    ).astype(jnp.bfloat16)
                    part_ref[rows, :] = jnp.dot(
                        xblk, w_in, preferred_element_type=jnp.float32
                    ).astype(jnp.bfloat16)
                    pending_rs.extend(rs_send(q) for q in grp)
            else:
                for p in range(N_DEV):
                    ag_recv_wait(p)
                out_ref[...] = jnp.dot(
                    h_ref[...], wouts[layer][...].astype(jnp.bfloat16),
                    preferred_element_type=jnp.float32,
                )

        for rdma in pending_ag:
            rdma.wait_send()

    return pl.pallas_call(
        body,
        out_shape=jax.ShapeDtypeStruct((b, d_shard), jnp.float32),
        in_specs=[pl.BlockSpec(memory_space=pltpu.VMEM)] * 7,
        out_specs=pl.BlockSpec(memory_space=pltpu.VMEM),
        scratch_shapes=[
            pltpu.VMEM((b, h_dim), jnp.bfloat16),
            pltpu.VMEM((N_DEV, CHUNK, h_dim), jnp.bfloat16),
            pltpu.VMEM((b, h_dim), jnp.bfloat16),
            pltpu.VMEM((CHUNK, h_dim), jnp.bfloat16),
            pltpu.SemaphoreType.DMA((N_DEV,)),
            pltpu.SemaphoreType.DMA((N_DEV,)),
            pltpu.SemaphoreType.DMA((N_DEV,)),
            pltpu.SemaphoreType.DMA((N_DEV,)),
        ],
    )(x, Win0, Wout0, Win1, Wout1, Win2, Wout2)
